# baseline (speedup 1.0000x reference)
import functools

import jax
import jax.numpy as jnp
from jax import lax
from jax.experimental import pallas as pl
from jax.experimental.pallas import tpu as pltpu
from jax.experimental.pallas import tpu_sc as plsc

_NBUF = 4
_C = 2


def _moe_kernel(idx_ref, x_ref, w_hbm, out_ref, w_buf, sem):
    E, D, O = w_hbm.shape
    DC = D // _C
    TOT = E * _C

    def make_copy(t, slot):
        e = t // _C
        c = jax.lax.rem(t, _C)
        return pltpu.make_async_copy(
            w_hbm.at[e, pl.ds(c * DC, DC), :],
            w_buf.at[slot],
            sem.at[slot],
        )

    for s in range(_NBUF):
        make_copy(s, s).start()

    def body(r, _):
        for s in range(_NBUF):
            t = r * _NBUF + s
            e = t // _C
            c = jax.lax.rem(t, _C)
            make_copy(t, s).wait()
            mask = (idx_ref[...] == e).astype(jnp.float32)
            xm = x_ref[:, pl.ds(c * DC, DC)] * mask
            part = jnp.dot(xm, w_buf[s], preferred_element_type=jnp.float32)

            @pl.when(t == 0)
            def _init():
                out_ref[...] = part

            @pl.when(t != 0)
            def _accum():
                out_ref[...] += part

            nxt = t + _NBUF

            @pl.when(nxt < TOT)
            def _prefetch():
                make_copy(nxt, s).start()

        return 0

    jax.lax.fori_loop(0, TOT // _NBUF, body, 0)


def _matmul_part(inputs, idx2d, w):
    N, D = inputs.shape
    E, _, O = w.shape
    return pl.pallas_call(
        _moe_kernel,
        in_specs=[
            pl.BlockSpec(memory_space=pltpu.VMEM),
            pl.BlockSpec(memory_space=pltpu.VMEM),
            pl.BlockSpec(memory_space=pl.ANY),
        ],
        out_specs=pl.BlockSpec(memory_space=pltpu.VMEM),
        out_shape=jax.ShapeDtypeStruct((N, O), jnp.float32),
        scratch_shapes=[
            pltpu.VMEM((_NBUF, D // _C, O), jnp.float32),
            pltpu.SemaphoreType.DMA((_NBUF,)),
        ],
    )(idx2d, inputs, w)


def _bias_part(b, indices):
    N = indices.shape[0]
    E, O = b.shape
    NW = 16
    per_w = N // NW
    mesh = plsc.VectorSubcoreMesh(core_axis_name="c", subcore_axis_name="s")

    @functools.partial(
        pl.kernel,
        out_type=jax.ShapeDtypeStruct((N, O), jnp.float32),
        mesh=mesh,
        scratch_types=[
            pltpu.VMEM((per_w,), jnp.int32),
            pltpu.VMEM((per_w, O), jnp.float32),
            pltpu.SemaphoreType.DMA,
        ],
    )
    def gather(b_hbm, idx_hbm, out_hbm, idx_v, rows_v, sem):
        wid = lax.axis_index("s") * 2 + lax.axis_index("c")

        @pl.when(wid < NW)
        def _():
            base = wid * per_w
            pltpu.sync_copy(idx_hbm.at[pl.ds(base, per_w)], idx_v)
            pltpu.async_copy(b_hbm.at[idx_v], rows_v, sem).wait()
            pltpu.sync_copy(rows_v, out_hbm.at[pl.ds(base, per_w)])

    return gather(b, indices)


def kernel(inputs, indices, w, b):
    N, _ = inputs.shape
    idx_i32 = indices.astype(jnp.int32)
    mm = _matmul_part(inputs, idx_i32.reshape(N, 1), w)
    bias = _bias_part(b, idx_i32)
    return mm + bias


# speedup vs baseline: 2.1086x; 2.1086x over previous
import jax
import jax.numpy as jnp
from jax.experimental import pallas as pl
from jax.experimental.pallas import tpu as pltpu

_NBUF = 4
_C = 2


def _moe_kernel(idx_ref, x_ref, w_hbm, b_ref, out_ref, w_buf, sem):
    E, D, O = w_hbm.shape
    DC = D // _C
    TOT = E * _C

    def make_copy(t, slot):
        e = t // _C
        c = jax.lax.rem(t, _C)
        return pltpu.make_async_copy(
            w_hbm.at[e, pl.ds(c * DC, DC), :],
            w_buf.at[slot],
            sem.at[slot],
        )

    for s in range(_NBUF):
        make_copy(s, s).start()

    def body(r, _):
        for s in range(_NBUF):
            t = r * _NBUF + s
            e = t // _C
            c = jax.lax.rem(t, _C)
            make_copy(t, s).wait()
            mask = (idx_ref[...] == e).astype(jnp.float32)
            xm = x_ref[:, pl.ds(c * DC, DC)] * mask
            part = jnp.dot(xm, w_buf[s], preferred_element_type=jnp.float32)
            part = jnp.where(c == _C - 1, part + mask * b_ref[e], part)

            @pl.when(t == 0)
            def _init():
                out_ref[...] = part

            @pl.when(t != 0)
            def _accum():
                out_ref[...] += part

            nxt = t + _NBUF

            @pl.when(nxt < TOT)
            def _prefetch():
                make_copy(nxt, s).start()

        return 0

    jax.lax.fori_loop(0, TOT // _NBUF, body, 0)


def kernel(inputs, indices, w, b):
    N, D = inputs.shape
    E, _, O = w.shape
    idx2d = indices.astype(jnp.int32).reshape(N, 1)
    b3d = b.reshape(E, 1, O)

    return pl.pallas_call(
        _moe_kernel,
        in_specs=[
            pl.BlockSpec(memory_space=pltpu.VMEM),
            pl.BlockSpec(memory_space=pltpu.VMEM),
            pl.BlockSpec(memory_space=pl.ANY),
            pl.BlockSpec(memory_space=pltpu.VMEM),
        ],
        out_specs=pl.BlockSpec(memory_space=pltpu.VMEM),
        out_shape=jax.ShapeDtypeStruct((N, O), jnp.float32),
        scratch_shapes=[
            pltpu.VMEM((_NBUF, D // _C, O), jnp.float32),
            pltpu.SemaphoreType.DMA((_NBUF,)),
        ],
    )(idx2d, inputs, w, b3d)
